# Initial kernel scaffold; baseline (speedup 1.0000x reference)
import jax
import jax.numpy as jnp
from jax.experimental import pallas as pl


def kernel(pe, token_positions):
    raise NotImplementedError("write your pallas kernel here")


# speedup vs baseline: 2.1294x; 2.1294x over previous
import functools

import jax
import jax.numpy as jnp
from jax import lax
from jax.experimental import pallas as pl
from jax.experimental.pallas import tpu as pltpu
from jax.experimental.pallas import tpu_sc as plsc


def _make_gather(n_rows, d_model, n_workers, num_cores, chunk):
    n_per_w = n_rows // n_workers
    n_chunks = n_per_w // chunk
    mesh = plsc.VectorSubcoreMesh(core_axis_name="c", subcore_axis_name="s")

    @functools.partial(
        pl.kernel,
        mesh=mesh,
        out_type=jax.ShapeDtypeStruct((n_rows, d_model), jnp.float32),
        scratch_types=[
            pltpu.VMEM((chunk,), jnp.int32),
            pltpu.VMEM((chunk, d_model), jnp.float32),
            pltpu.SemaphoreType.DMA,
        ],
    )
    def gather_kernel(table_hbm, idx_hbm, out_hbm, idx_v, rows_v, sem):
        wid = lax.axis_index("s") * num_cores + lax.axis_index("c")
        base = wid * n_per_w

        def body(i, carry):
            off = base + i * chunk
            pltpu.sync_copy(idx_hbm.at[pl.ds(off, chunk)], idx_v)
            pltpu.async_copy(table_hbm.at[idx_v], rows_v, sem).wait()
            pltpu.sync_copy(rows_v, out_hbm.at[pl.ds(off, chunk)])
            return carry

        lax.fori_loop(0, n_chunks, body, 0)

    return gather_kernel


def kernel(pe, token_positions):
    batch, seq_len = token_positions.shape
    max_seq_len, d_model = pe.shape
    n_rows = batch * seq_len
    idx = token_positions.reshape(n_rows)

    info = plsc.get_sparse_core_info()
    n_workers = info.num_cores * info.num_subcores
    gather = _make_gather(n_rows, d_model, n_workers, info.num_cores, chunk=64)
    out = gather(pe, idx)
    return out.reshape(batch, seq_len, d_model)


# speedup vs baseline: 2.3721x; 1.1140x over previous
import functools

import jax
import jax.numpy as jnp
from jax import lax
from jax.experimental import pallas as pl
from jax.experimental.pallas import tpu as pltpu
from jax.experimental.pallas import tpu_sc as plsc

_NBUF = 4
_AHEAD = 2


def _make_gather(n_rows, d_model, n_workers, num_cores, chunk):
    n_per_w = n_rows // n_workers
    n_chunks = n_per_w // chunk
    assert n_per_w % chunk == 0 and n_chunks % _NBUF == 0
    n_groups = n_chunks // _NBUF
    mesh = plsc.VectorSubcoreMesh(core_axis_name="c", subcore_axis_name="s")

    @functools.partial(
        pl.kernel,
        mesh=mesh,
        out_type=jax.ShapeDtypeStruct((n_rows, d_model), jnp.float32),
        scratch_types=[
            pltpu.VMEM((n_chunks, chunk), jnp.int32),
            pltpu.VMEM((_NBUF, chunk, d_model), jnp.float32),
        ]
        + [pltpu.SemaphoreType.DMA] * (2 * _NBUF),
    )
    def gather_kernel(table_hbm, idx_hbm, out_hbm, idx_v, rows_v, *sems):
        gsem = sems[:_NBUF]
        ssem = sems[_NBUF:]
        wid = lax.axis_index("s") * num_cores + lax.axis_index("c")
        base = wid * n_per_w

        pltpu.sync_copy(idx_hbm.at[wid], idx_v)

        def start_gather(b, g):
            pltpu.async_copy(table_hbm.at[idx_v.at[g]], rows_v.at[b], gsem[b])

        def wait_gather(b, g):
            pltpu.make_async_copy(
                table_hbm.at[idx_v.at[g]], rows_v.at[b], gsem[b]
            ).wait()

        def start_scatter(b, g):
            pltpu.async_copy(
                rows_v.at[b], out_hbm.at[pl.ds(base + g * chunk, chunk)], ssem[b]
            )

        def wait_scatter(b):
            pltpu.make_async_copy(
                rows_v.at[b], out_hbm.at[pl.ds(base, chunk)], ssem[b]
            ).wait()

        for b in range(_AHEAD):
            start_gather(b, b)

        def group(o, carry):
            for j in range(_NBUF):
                g = o * _NBUF + j
                bn = (j + _AHEAD) % _NBUF
                wait_gather(j, g)
                start_scatter(j, g)

                @pl.when(g + _AHEAD < n_chunks)
                def _():
                    @pl.when(g >= _NBUF - _AHEAD)
                    def _():
                        wait_scatter(bn)

                    start_gather(bn, g + _AHEAD)

            return carry

        lax.fori_loop(0, n_groups, group, 0)

        for j in range(_NBUF):
            wait_scatter(j)

    return gather_kernel


def kernel(pe, token_positions):
    batch, seq_len = token_positions.shape
    max_seq_len, d_model = pe.shape
    n_rows = batch * seq_len

    info = plsc.get_sparse_core_info()
    n_workers = info.num_cores * info.num_subcores
    chunk = 16
    n_per_w = n_rows // n_workers
    idx = token_positions.reshape(n_workers, n_per_w // chunk, chunk)

    gather = _make_gather(n_rows, d_model, n_workers, info.num_cores, chunk)
    out = gather(pe, idx)
    return out.reshape(batch, seq_len, d_model)
